# Initial kernel scaffold; baseline (speedup 1.0000x reference)
import jax
import jax.numpy as jnp
from jax.experimental import pallas as pl


def kernel(input_ids, token_type_ids, ner_ids, ent_ids, word_emb, pos_emb, type_emb, ner_emb, ent_emb, gamma, beta):
    raise NotImplementedError("write your pallas kernel here")


# speedup vs baseline: 3.4203x; 3.4203x over previous
import functools

import jax
import jax.numpy as jnp
from jax import lax
from jax.experimental import pallas as pl
from jax.experimental.pallas import tpu as pltpu
from jax.experimental.pallas import tpu_sc as plsc

VOCAB = 100000
HID = 768
MAXPOS = 512
TYPES = 2
NER = 7
ENT = 43
B = 64
S = 512
EPS = 1e-12

NTOK = B * S

NC = 2
NS = 16
NW = NC * NS
TOK_PER_W = NTOK // NW
CHUNK = 128
NSTEP = TOK_PER_W // CHUNK


def _sc_gather_body(table_hbm, idx_hbm, out_hbm, idx_v, rows_v, sem):
    wid = lax.axis_index("s") * NC + lax.axis_index("c")
    base = wid * TOK_PER_W
    pltpu.sync_copy(idx_hbm.at[pl.ds(base, TOK_PER_W)], idx_v)
    for j in range(NSTEP):
        pltpu.async_copy(
            table_hbm.at[idx_v.at[pl.ds(j * CHUNK, CHUNK)]], rows_v, sem
        ).wait()
        pltpu.sync_copy(rows_v, out_hbm.at[pl.ds(base + j * CHUNK, CHUNK)])


@functools.cache
def _sc_gather():
    return pl.kernel(
        _sc_gather_body,
        out_type=jax.ShapeDtypeStruct((NTOK, HID), jnp.float32),
        mesh=plsc.VectorSubcoreMesh(
            core_axis_name="c", subcore_axis_name="s",
            num_cores=NC, num_subcores=NS,
        ),
        scratch_types=[
            pltpu.VMEM((TOK_PER_W,), jnp.int32),
            pltpu.VMEM((CHUNK, HID), jnp.float32),
            pltpu.SemaphoreType.DMA,
        ],
    )


TBLK = 256
NBLK = NTOK // TBLK
POS_BLKS = MAXPOS // TBLK
NSMALL = TYPES + NER + ENT
NSMALL_PAD = 56


def _tc_finish_body(g_ref, tt_ref, nr_ref, en_ref, pos_ref, tab_ref,
                    gamma_ref, beta_ref, o_ref):
    tt = tt_ref[0, 0, :]
    nr = nr_ref[0, 0, :]
    en = en_ref[0, 0, :]
    lane = lax.broadcasted_iota(jnp.int32, (TBLK, NSMALL_PAD), 1)
    oh = (
        (lane == tt[:, None])
        | (lane == nr[:, None] + TYPES)
        | (lane == en[:, None] + TYPES + NER)
    )
    small = jnp.dot(
        oh.astype(jnp.float32), tab_ref[...], preferred_element_type=jnp.float32
    )
    e = g_ref[...] + pos_ref[...] + small
    mu = jnp.mean(e, axis=-1, keepdims=True)
    d = e - mu
    var = jnp.mean(d * d, axis=-1, keepdims=True)
    o_ref[...] = d * lax.rsqrt(var + EPS) * gamma_ref[...] + beta_ref[...]


_tc_finish = pl.pallas_call(
    _tc_finish_body,
    grid=(NBLK,),
    in_specs=[
        pl.BlockSpec((TBLK, HID), lambda i: (i, 0)),
        pl.BlockSpec((1, 1, TBLK), lambda i: (i, 0, 0)),
        pl.BlockSpec((1, 1, TBLK), lambda i: (i, 0, 0)),
        pl.BlockSpec((1, 1, TBLK), lambda i: (i, 0, 0)),
        pl.BlockSpec((TBLK, HID), lambda i: (i % POS_BLKS, 0)),
        pl.BlockSpec((NSMALL_PAD, HID), lambda i: (0, 0)),
        pl.BlockSpec((1, HID), lambda i: (0, 0)),
        pl.BlockSpec((1, HID), lambda i: (0, 0)),
    ],
    out_specs=pl.BlockSpec((TBLK, HID), lambda i: (i, 0)),
    out_shape=jax.ShapeDtypeStruct((NTOK, HID), jnp.float32),
)


def kernel(input_ids, token_type_ids, ner_ids, ent_ids, word_emb, pos_emb,
           type_emb, ner_emb, ent_emb, gamma, beta):
    ids = input_ids.reshape(NTOK).astype(jnp.int32)
    gathered = _sc_gather()(word_emb, ids)

    tt = token_type_ids.reshape(NBLK, 1, TBLK).astype(jnp.int32)
    nr = ner_ids.reshape(NBLK, 1, TBLK).astype(jnp.int32)
    en = ent_ids.reshape(NBLK, 1, TBLK).astype(jnp.int32)
    tab = jnp.zeros((NSMALL_PAD, HID), jnp.float32)
    tab = tab.at[:NSMALL].set(jnp.concatenate([type_emb, ner_emb, ent_emb], 0))
    out = _tc_finish(gathered, tt, nr, en, pos_emb,
                     tab, gamma.reshape(1, HID), beta.reshape(1, HID))
    return out.reshape(B, S, HID)


# speedup vs baseline: 3.4450x; 1.0072x over previous
import functools

import jax
import jax.numpy as jnp
from jax import lax
from jax.experimental import pallas as pl
from jax.experimental.pallas import tpu as pltpu
from jax.experimental.pallas import tpu_sc as plsc

VOCAB = 100000
HID = 768
MAXPOS = 512
TYPES = 2
NER = 7
ENT = 43
B = 64
S = 512
EPS = 1e-12

NTOK = B * S

NC = 2
NS = 16
NW = NC * NS
TOK_PER_W = NTOK // NW
CHUNK = 64
NSTEP = TOK_PER_W // CHUNK


def _sc_gather_body(table_hbm, idx_hbm, out_hbm, idx_v, rows0, rows1,
                    gsem0, gsem1, wsem0, wsem1):
    wid = lax.axis_index("s") * NC + lax.axis_index("c")
    base = wid * TOK_PER_W
    pltpu.sync_copy(idx_hbm.at[pl.ds(base, TOK_PER_W)], idx_v)
    bufs = (rows0, rows1)
    gsems = (gsem0, gsem1)
    wsems = (wsem0, wsem1)

    def gather(j):
        return pltpu.async_copy(
            table_hbm.at[idx_v.at[pl.ds(j * CHUNK, CHUNK)]],
            bufs[j % 2], gsems[j % 2],
        )

    def writeback(j):
        return pltpu.async_copy(
            bufs[j % 2], out_hbm.at[pl.ds(base + j * CHUNK, CHUNK)],
            wsems[j % 2],
        )

    g = gather(0)
    writes = [None, None]
    for j in range(NSTEP):
        if j + 1 < NSTEP:
            if writes[(j + 1) % 2] is not None:
                writes[(j + 1) % 2].wait()
            g_next = gather(j + 1)
        g.wait()
        writes[j % 2] = writeback(j)
        if j + 1 < NSTEP:
            g = g_next
    writes[(NSTEP - 2) % 2].wait()
    writes[(NSTEP - 1) % 2].wait()


@functools.cache
def _sc_gather():
    return pl.kernel(
        _sc_gather_body,
        out_type=jax.ShapeDtypeStruct((NTOK, HID), jnp.float32),
        mesh=plsc.VectorSubcoreMesh(
            core_axis_name="c", subcore_axis_name="s",
            num_cores=NC, num_subcores=NS,
        ),
        scratch_types=[
            pltpu.VMEM((TOK_PER_W,), jnp.int32),
            pltpu.VMEM((CHUNK, HID), jnp.float32),
            pltpu.VMEM((CHUNK, HID), jnp.float32),
            pltpu.SemaphoreType.DMA,
            pltpu.SemaphoreType.DMA,
            pltpu.SemaphoreType.DMA,
            pltpu.SemaphoreType.DMA,
        ],
    )


TBLK = 256
NBLK = NTOK // TBLK
POS_BLKS = MAXPOS // TBLK
NSMALL = TYPES + NER + ENT
NSMALL_PAD = 56


def _tc_finish_body(g_ref, tt_ref, nr_ref, en_ref, pos_ref, tab_ref,
                    gamma_ref, beta_ref, o_ref):
    tt = tt_ref[0, 0, :]
    nr = nr_ref[0, 0, :]
    en = en_ref[0, 0, :]
    lane = lax.broadcasted_iota(jnp.int32, (TBLK, NSMALL_PAD), 1)
    oh = (
        (lane == tt[:, None])
        | (lane == nr[:, None] + TYPES)
        | (lane == en[:, None] + TYPES + NER)
    )
    small = jnp.dot(
        oh.astype(jnp.float32), tab_ref[...], preferred_element_type=jnp.float32
    )
    e = g_ref[...] + pos_ref[...] + small
    mu = jnp.mean(e, axis=-1, keepdims=True)
    d = e - mu
    var = jnp.mean(d * d, axis=-1, keepdims=True)
    o_ref[...] = d * lax.rsqrt(var + EPS) * gamma_ref[...] + beta_ref[...]


_tc_finish = pl.pallas_call(
    _tc_finish_body,
    grid=(NBLK,),
    in_specs=[
        pl.BlockSpec((TBLK, HID), lambda i: (i, 0)),
        pl.BlockSpec((1, 1, TBLK), lambda i: (i, 0, 0)),
        pl.BlockSpec((1, 1, TBLK), lambda i: (i, 0, 0)),
        pl.BlockSpec((1, 1, TBLK), lambda i: (i, 0, 0)),
        pl.BlockSpec((TBLK, HID), lambda i: (i % POS_BLKS, 0)),
        pl.BlockSpec((NSMALL_PAD, HID), lambda i: (0, 0)),
        pl.BlockSpec((1, HID), lambda i: (0, 0)),
        pl.BlockSpec((1, HID), lambda i: (0, 0)),
    ],
    out_specs=pl.BlockSpec((TBLK, HID), lambda i: (i, 0)),
    out_shape=jax.ShapeDtypeStruct((NTOK, HID), jnp.float32),
)


def kernel(input_ids, token_type_ids, ner_ids, ent_ids, word_emb, pos_emb,
           type_emb, ner_emb, ent_emb, gamma, beta):
    ids = input_ids.reshape(NTOK).astype(jnp.int32)
    gathered = _sc_gather()(word_emb, ids)

    tt = token_type_ids.reshape(NBLK, 1, TBLK).astype(jnp.int32)
    nr = ner_ids.reshape(NBLK, 1, TBLK).astype(jnp.int32)
    en = ent_ids.reshape(NBLK, 1, TBLK).astype(jnp.int32)
    tab = jnp.zeros((NSMALL_PAD, HID), jnp.float32)
    tab = tab.at[:NSMALL].set(jnp.concatenate([type_emb, ner_emb, ent_emb], 0))
    out = _tc_finish(gathered, tt, nr, en, pos_emb,
                     tab, gamma.reshape(1, HID), beta.reshape(1, HID))
    return out.reshape(B, S, HID)


# speedup vs baseline: 4.4938x; 1.3045x over previous
import functools

import jax
import jax.numpy as jnp
from jax import lax
from jax.experimental import pallas as pl
from jax.experimental.pallas import tpu as pltpu
from jax.experimental.pallas import tpu_sc as plsc

VOCAB = 100000
HID = 768
MAXPOS = 512
TYPES = 2
NER = 7
ENT = 43
B = 64
S = 512
EPS = 1e-12

NTOK = B * S

NC = 2
NS = 16
NW = NC * NS
TOK_PER_W = NTOK // NW
CHUNK = 64
NSTEP = TOK_PER_W // CHUNK


def _sc_gather_body(table_hbm, idx_hbm, out_hbm, idx_v, rows0, rows1,
                    gsem0, gsem1, wsem0, wsem1):
    wid = lax.axis_index("s") * NC + lax.axis_index("c")
    base = wid * TOK_PER_W
    pltpu.sync_copy(idx_hbm.at[pl.ds(base, TOK_PER_W)], idx_v)
    bufs = (rows0, rows1)
    gsems = (gsem0, gsem1)
    wsems = (wsem0, wsem1)

    def gather(j):
        return pltpu.async_copy(
            table_hbm.at[idx_v.at[pl.ds(j * CHUNK, CHUNK)]],
            bufs[j % 2], gsems[j % 2],
        )

    def writeback(j):
        return pltpu.async_copy(
            bufs[j % 2], out_hbm.at[pl.ds(base + j * CHUNK, CHUNK)],
            wsems[j % 2],
        )

    g = gather(0)
    writes = [None, None]
    for j in range(NSTEP):
        if j + 1 < NSTEP:
            if writes[(j + 1) % 2] is not None:
                writes[(j + 1) % 2].wait()
            g_next = gather(j + 1)
        g.wait()
        writes[j % 2] = writeback(j)
        if j + 1 < NSTEP:
            g = g_next
    writes[(NSTEP - 2) % 2].wait()
    writes[(NSTEP - 1) % 2].wait()


@functools.cache
def _sc_gather():
    return pl.kernel(
        _sc_gather_body,
        out_type=jax.ShapeDtypeStruct((NTOK, HID), jnp.float32),
        mesh=plsc.VectorSubcoreMesh(
            core_axis_name="c", subcore_axis_name="s",
            num_cores=NC, num_subcores=NS,
        ),
        scratch_types=[
            pltpu.VMEM((TOK_PER_W,), jnp.int32),
            pltpu.VMEM((CHUNK, HID), jnp.float32),
            pltpu.VMEM((CHUNK, HID), jnp.float32),
            pltpu.SemaphoreType.DMA,
            pltpu.SemaphoreType.DMA,
            pltpu.SemaphoreType.DMA,
            pltpu.SemaphoreType.DMA,
        ],
    )


TBLK = 512
NBLK = NTOK // TBLK
NSMALL = TYPES + NER + ENT
NSMALL_PAD = 56


def _tc_finish_body(g_ref, tt_ref, nr_ref, en_ref, pos_ref, tab_ref,
                    gamma_ref, beta_ref, o_ref):
    tt = tt_ref[0, 0, :]
    nr = nr_ref[0, 0, :]
    en = en_ref[0, 0, :]
    lane = lax.broadcasted_iota(jnp.int32, (TBLK, NSMALL_PAD), 1)
    oh = (
        (lane == tt[:, None])
        | (lane == nr[:, None] + TYPES)
        | (lane == en[:, None] + TYPES + NER)
    )
    small = jnp.dot(
        oh.astype(jnp.float32), tab_ref[...], preferred_element_type=jnp.float32
    )
    e = g_ref[...] + pos_ref[...] + small
    mu = jnp.mean(e, axis=-1, keepdims=True)
    d = e - mu
    var = jnp.mean(d * d, axis=-1, keepdims=True)
    o_ref[...] = d * lax.rsqrt(var + EPS) * gamma_ref[...] + beta_ref[...]


_tc_finish = pl.pallas_call(
    _tc_finish_body,
    grid=(NBLK,),
    in_specs=[
        pl.BlockSpec((TBLK, HID), lambda i: (i, 0)),
        pl.BlockSpec((1, 1, TBLK), lambda i: (i, 0, 0)),
        pl.BlockSpec((1, 1, TBLK), lambda i: (i, 0, 0)),
        pl.BlockSpec((1, 1, TBLK), lambda i: (i, 0, 0)),
        pl.BlockSpec((TBLK, HID), lambda i: (0, 0)),
        pl.BlockSpec((NSMALL_PAD, HID), lambda i: (0, 0)),
        pl.BlockSpec((1, HID), lambda i: (0, 0)),
        pl.BlockSpec((1, HID), lambda i: (0, 0)),
    ],
    out_specs=pl.BlockSpec((TBLK, HID), lambda i: (i, 0)),
    out_shape=jax.ShapeDtypeStruct((NTOK, HID), jnp.float32),
)


def kernel(input_ids, token_type_ids, ner_ids, ent_ids, word_emb, pos_emb,
           type_emb, ner_emb, ent_emb, gamma, beta):
    ids = input_ids.reshape(NTOK).astype(jnp.int32)
    gathered = _sc_gather()(word_emb, ids)

    tt = token_type_ids.reshape(NBLK, 1, TBLK).astype(jnp.int32)
    nr = ner_ids.reshape(NBLK, 1, TBLK).astype(jnp.int32)
    en = ent_ids.reshape(NBLK, 1, TBLK).astype(jnp.int32)
    tab = jnp.zeros((NSMALL_PAD, HID), jnp.float32)
    tab = tab.at[:NSMALL].set(jnp.concatenate([type_emb, ner_emb, ent_emb], 0))
    out = _tc_finish(gathered, tt, nr, en, pos_emb,
                     tab, gamma.reshape(1, HID), beta.reshape(1, HID))
    return out.reshape(B, S, HID)
